# Initial kernel scaffold; baseline (speedup 1.0000x reference)
import jax
import jax.numpy as jnp
from jax.experimental import pallas as pl


def kernel(x, edge_index, Wproj, bproj, W1, W2):
    raise NotImplementedError("write your pallas kernel here")


# speedup vs baseline: 5.8044x; 5.8044x over previous
import functools
import math

import jax
import jax.numpy as jnp
from jax import lax
from jax.experimental import pallas as pl
from jax.experimental.pallas import tpu as pltpu
from jax.experimental.pallas import tpu_sc as plsc

ALPHA = 0.5
THETA = 1.0
N_LAYERS = 4

NC = 2
NS = 16
NW = NC * NS
W = 128

_MESH = plsc.VectorSubcoreMesh(core_axis_name="c", subcore_axis_name="s")


def _sc_propagate(n, d, npad, nwin, rpt):
    @functools.partial(
        pl.kernel,
        mesh=_MESH,
        out_type=jax.ShapeDtypeStruct((NC * npad, d), jnp.float32),
        scratch_types=[
            pltpu.VMEM((W,), jnp.int32),
            pltpu.VMEM((W,), jnp.int32),
            pltpu.VMEM((W,), jnp.int32),
            pltpu.VMEM((W,), jnp.int32),
            pltpu.VMEM((W, d), jnp.float32),
            pltpu.VMEM((W, d), jnp.float32),
            pltpu.VMEM_SHARED((npad, d), jnp.float32),
            pltpu.SemaphoreType.DMA,
            pltpu.SemaphoreType.DMA,
            pltpu.SemaphoreType.DMA,
            pltpu.SemaphoreType.DMA,
        ],
    )
    def prop_kernel(g_hbm, src_hbm, dst_hbm, zeros_hbm, out_hbm,
                    si0, si1, di0, di1, rows0, rows1, acc,
                    sem0, sem1, semi0, semi1):
        cid = lax.axis_index("c")
        sid = lax.axis_index("s")
        wid = cid * NS + sid
        pltpu.sync_copy(zeros_hbm, rows0)

        @pl.loop(0, rpt // W)
        def _(k):
            pltpu.sync_copy(rows0, acc.at[pl.ds(sid * rpt + k * W, W)])

        pltpu.sync_copy(src_hbm.at[wid * nwin + 0], si0)
        pltpu.sync_copy(dst_hbm.at[wid * nwin + 0], di0)
        pltpu.make_async_copy(src_hbm.at[wid * nwin + 1], si1, semi1).start()
        pltpu.make_async_copy(dst_hbm.at[wid * nwin + 1], di1, semi1).start()
        plsc.subcore_barrier()

        pltpu.make_async_copy(g_hbm.at[si0], rows0, sem0).start()

        @pl.loop(0, nwin, step=2)
        def _(j):
            pltpu.make_async_copy(src_hbm.at[wid * nwin + j + 1], si1, semi1).wait()
            pltpu.make_async_copy(dst_hbm.at[wid * nwin + j + 1], di1, semi1).wait()
            pltpu.make_async_copy(g_hbm.at[si1], rows1, sem1).start()
            pltpu.make_async_copy(g_hbm.at[si0], rows0, sem0).wait()
            pltpu.sync_copy(rows0, acc.at[di0], add=True)

            @pl.when(j + 2 < nwin)
            def _():
                pltpu.make_async_copy(src_hbm.at[wid * nwin + j + 2], si0, semi0).start()
                pltpu.make_async_copy(dst_hbm.at[wid * nwin + j + 2], di0, semi0).start()
                pltpu.make_async_copy(src_hbm.at[wid * nwin + j + 2], si0, semi0).wait()
                pltpu.make_async_copy(dst_hbm.at[wid * nwin + j + 2], di0, semi0).wait()
                pltpu.make_async_copy(g_hbm.at[si0], rows0, sem0).start()

            pltpu.make_async_copy(g_hbm.at[si1], rows1, sem1).wait()
            pltpu.sync_copy(rows1, acc.at[di1], add=True)

            @pl.when(j + 3 < nwin)
            def _():
                pltpu.make_async_copy(src_hbm.at[wid * nwin + j + 3], si1, semi1).start()
                pltpu.make_async_copy(dst_hbm.at[wid * nwin + j + 3], di1, semi1).start()

        plsc.subcore_barrier()

        nchunk = rpt // W
        for k in range(nchunk):
            buf = rows0 if k % 2 == 0 else rows1
            sem = sem0 if k % 2 == 0 else sem1
            r0 = sid * rpt + k * W
            if k >= 2:
                rp = sid * rpt + (k - 2) * W
                pltpu.make_async_copy(buf, out_hbm.at[pl.ds(cid * npad + rp, W)], sem).wait()
            pltpu.sync_copy(acc.at[pl.ds(r0, W)], buf)
            pltpu.make_async_copy(buf, out_hbm.at[pl.ds(cid * npad + r0, W)], sem).start()
        for k in range(max(nchunk - 2, 0), nchunk):
            buf = rows0 if k % 2 == 0 else rows1
            sem = sem0 if k % 2 == 0 else sem1
            r0 = sid * rpt + k * W
            pltpu.make_async_copy(buf, out_hbm.at[pl.ds(cid * npad + r0, W)], sem).wait()

    return prop_kernel


_HI = jax.lax.Precision.HIGHEST


def _tc_proj(n, d, rb):
    def body(x_ref, wp_ref, b_ref, w2e_ref, h0_ref, c_ref):
        h0 = jnp.dot(x_ref[...], wp_ref[...],
                     preferred_element_type=jnp.float32, precision=_HI)
        h0 = h0 + b_ref[...]
        h0_ref[...] = h0
        for i in range(N_LAYERS):
            c_ref[i] = jnp.dot(h0, w2e_ref[i],
                               preferred_element_type=jnp.float32, precision=_HI)

    grid = n // rb
    return pl.pallas_call(
        body,
        grid=(grid,),
        in_specs=[
            pl.BlockSpec((rb, d), lambda i: (i, 0)),
            pl.BlockSpec((d, d), lambda i: (0, 0)),
            pl.BlockSpec((1, d), lambda i: (0, 0)),
            pl.BlockSpec((N_LAYERS, d, d), lambda i: (0, 0, 0)),
        ],
        out_specs=[
            pl.BlockSpec((rb, d), lambda i: (i, 0)),
            pl.BlockSpec((N_LAYERS, rb, d), lambda i: (0, i, 0)),
        ],
        out_shape=[
            jax.ShapeDtypeStruct((n, d), jnp.float32),
            jax.ShapeDtypeStruct((N_LAYERS, n, d), jnp.float32),
        ],
    )


def _tc_scale(n, d, npad, rb):
    def body(deg_ref, h0_ref, g_ref, dis_ref):
        dg = 1.0 + deg_ref[0, :, :1] + deg_ref[1, :, :1]
        dis = jax.lax.rsqrt(dg)
        g_ref[...] = h0_ref[...] * dis
        dis_ref[...] = jnp.broadcast_to(dis, (rb, d))

    return pl.pallas_call(
        body,
        grid=(n // rb,),
        in_specs=[
            pl.BlockSpec((NC, rb, d), lambda i: (0, i, 0)),
            pl.BlockSpec((rb, d), lambda i: (i, 0)),
        ],
        out_specs=[
            pl.BlockSpec((rb, d), lambda i: (i, 0)),
            pl.BlockSpec((rb, d), lambda i: (i, 0)),
        ],
        out_shape=[
            jax.ShapeDtypeStruct((n, d), jnp.float32),
            jax.ShapeDtypeStruct((n, d), jnp.float32),
        ],
    )


def _tc_layer(n, d, npad, rb, emit_g):
    def body(s_ref, g_ref, dis_ref, w1e_ref, c_ref, *out_refs):
        dis = dis_ref[...]
        u = (s_ref[0] + s_ref[1] + g_ref[...]) * dis
        h = jnp.dot(u, w1e_ref[...],
                    preferred_element_type=jnp.float32, precision=_HI)
        h = jnp.maximum(h + c_ref[...], 0.0)
        if emit_g:
            out_refs[0][...] = h * dis
        else:
            out_refs[0][...] = h

    return pl.pallas_call(
        body,
        grid=(n // rb,),
        in_specs=[
            pl.BlockSpec((NC, rb, d), lambda i: (0, i, 0)),
            pl.BlockSpec((rb, d), lambda i: (i, 0)),
            pl.BlockSpec((rb, d), lambda i: (i, 0)),
            pl.BlockSpec((d, d), lambda i: (0, 0)),
            pl.BlockSpec((rb, d), lambda i: (i, 0)),
        ],
        out_specs=pl.BlockSpec((rb, d), lambda i: (i, 0)),
        out_shape=jax.ShapeDtypeStruct((n, d), jnp.float32),
    )


def kernel(x, edge_index, Wproj, bproj, W1, W2):
    n, d = x.shape
    e = edge_index.shape[1]

    ept = -(-e // NW)
    nwin = -(-ept // W)
    nwin += nwin % 2
    epad = NW * nwin * W
    rpt = -(-(n + 1) // (NS * W)) * W
    npad = NS * rpt
    rb = 1000
    assert n % rb == 0

    src = edge_index[0].astype(jnp.int32)
    dst = edge_index[1].astype(jnp.int32)
    pad = epad - e
    srcp = jnp.concatenate([src, jnp.zeros((pad,), jnp.int32)]).reshape(NW * nwin, W)
    dstp = jnp.concatenate([dst, jnp.full((pad,), n, jnp.int32)]).reshape(NW * nwin, W)

    eye = jnp.eye(d, dtype=jnp.float32)
    w1e, w2e = [], []
    for i in range(N_LAYERS):
        beta = math.log(THETA / (i + 1) + 1.0)
        w1e.append((1.0 - ALPHA) * ((1.0 - beta) * eye + beta * W1[i]))
        w2e.append(ALPHA * ((1.0 - beta) * eye + beta * W2[i]))
    w1e = jnp.stack(w1e)
    w2e = jnp.stack(w2e)

    zd = jnp.zeros((W, d), jnp.float32)
    onesnd = jnp.ones((n, d), jnp.float32)

    prop = _sc_propagate(n, d, npad, nwin, rpt)

    degd = prop(onesnd, srcp, dstp, zd).reshape(NC, npad, d)
    h0, c = _tc_proj(n, d, rb)(x, Wproj, bproj.reshape(1, d), w2e)
    g, disrep = _tc_scale(n, d, npad, rb)(degd, h0)

    layer_mid = _tc_layer(n, d, npad, rb, emit_g=True)
    layer_last = _tc_layer(n, d, npad, rb, emit_g=False)
    for i in range(N_LAYERS):
        s = prop(g, srcp, dstp, zd).reshape(NC, npad, d)
        if i + 1 < N_LAYERS:
            g = layer_mid(s, g, disrep, w1e[i], c[i])
        else:
            h = layer_last(s, g, disrep, w1e[i], c[i])
    return h


# speedup vs baseline: 5.8145x; 1.0017x over previous
import functools
import math

import jax
import jax.numpy as jnp
from jax import lax
from jax.experimental import pallas as pl
from jax.experimental.pallas import tpu as pltpu
from jax.experimental.pallas import tpu_sc as plsc

ALPHA = 0.5
THETA = 1.0
N_LAYERS = 4

NC = 2
NS = 16
NW = NC * NS
W = 128

_MESH = plsc.VectorSubcoreMesh(core_axis_name="c", subcore_axis_name="s")


def _sc_propagate(n, d, npad, nwin, rpt):
    @functools.partial(
        pl.kernel,
        mesh=_MESH,
        out_type=jax.ShapeDtypeStruct((NC * npad, d), jnp.float32),
        scratch_types=[
            pltpu.VMEM((2, W), jnp.int32),
            pltpu.VMEM((2, W), jnp.int32),
            pltpu.VMEM((2, W), jnp.int32),
            pltpu.VMEM((2, W), jnp.int32),
            pltpu.VMEM((2, W), jnp.int32),
            pltpu.VMEM((2, W), jnp.int32),
            pltpu.VMEM((2, W), jnp.int32),
            pltpu.VMEM((2, W), jnp.int32),
            pltpu.VMEM((W, d), jnp.float32),
            pltpu.VMEM((W, d), jnp.float32),
            pltpu.VMEM_SHARED((npad, d), jnp.float32),
            pltpu.SemaphoreType.DMA,
            pltpu.SemaphoreType.DMA,
            pltpu.SemaphoreType.DMA,
            pltpu.SemaphoreType.DMA,
            pltpu.SemaphoreType.DMA,
            pltpu.SemaphoreType.DMA,
            pltpu.SemaphoreType.DMA,
            pltpu.SemaphoreType.DMA,
            pltpu.SemaphoreType.DMA,
            pltpu.SemaphoreType.DMA,
            pltpu.SemaphoreType.DMA,
            pltpu.SemaphoreType.DMA,
        ],
    )
    def prop_kernel(g_hbm, sd_hbm, zeros_hbm, out_hbm,
                    x0, x1, x2, x3, x4, x5, x6, x7, rows0, rows1, acc,
                    i0, i1, i2, i3, i4, i5, i6, i7,
                    sg0, sg1, ss0, ss1):
        slots = (x0, x1, x2, x3, x4, x5, x6, x7)
        isems = (i0, i1, i2, i3, i4, i5, i6, i7)
        rows = (rows0, rows1)
        gsems = (sg0, sg1)
        ssems = (ss0, ss1)
        cid = lax.axis_index("c")
        sid = lax.axis_index("s")
        wid = cid * NS + sid
        base = wid * nwin

        pltpu.sync_copy(zeros_hbm, rows0)

        @pl.loop(0, rpt // W)
        def _(k):
            pltpu.sync_copy(rows0, acc.at[pl.ds(sid * rpt + k * W, W)])

        for k in range(6):
            pltpu.make_async_copy(sd_hbm.at[base + k], slots[k], isems[k]).start()
        pltpu.make_async_copy(sd_hbm.at[base + 0], slots[0], isems[0]).wait()
        pltpu.make_async_copy(g_hbm.at[slots[0].at[0]], rows0, sg0).start()
        pltpu.make_async_copy(sd_hbm.at[base + 1], slots[1], isems[1]).wait()
        pltpu.make_async_copy(g_hbm.at[slots[1].at[0]], rows1, sg1).start()
        plsc.subcore_barrier()

        @pl.loop(0, nwin, step=8)
        def _(j):
            for k in range(0, 8, 2):
                sa, sb = slots[k], slots[k + 1]
                pltpu.make_async_copy(g_hbm.at[sa.at[0]], rows0, sg0).wait()
                pltpu.make_async_copy(
                    rows0, acc.at[sa.at[1]], ss0).start(add=True)
                pltpu.make_async_copy(g_hbm.at[sb.at[0]], rows1, sg1).wait()
                pltpu.make_async_copy(
                    rows1, acc.at[sb.at[1]], ss1).start(add=True)

                @pl.when(j + k + 6 < nwin)
                def _():
                    pltpu.make_async_copy(
                        sd_hbm.at[base + j + k + 6],
                        slots[(k + 6) % 8], isems[(k + 6) % 8]).start()

                @pl.when(j + k + 7 < nwin)
                def _():
                    pltpu.make_async_copy(
                        sd_hbm.at[base + j + k + 7],
                        slots[(k + 7) % 8], isems[(k + 7) % 8]).start()

                pltpu.make_async_copy(rows0, acc.at[sa.at[1]], ss0).wait()

                @pl.when(j + k + 2 < nwin)
                def _():
                    pltpu.make_async_copy(
                        sd_hbm.at[base + j + k + 2],
                        slots[(k + 2) % 8], isems[(k + 2) % 8]).wait()
                    pltpu.make_async_copy(
                        g_hbm.at[slots[(k + 2) % 8].at[0]], rows0, sg0).start()

                pltpu.make_async_copy(rows1, acc.at[sb.at[1]], ss1).wait()

                @pl.when(j + k + 3 < nwin)
                def _():
                    pltpu.make_async_copy(
                        sd_hbm.at[base + j + k + 3],
                        slots[(k + 3) % 8], isems[(k + 3) % 8]).wait()
                    pltpu.make_async_copy(
                        g_hbm.at[slots[(k + 3) % 8].at[0]], rows1, sg1).start()

        plsc.subcore_barrier()

        nchunk = rpt // W
        for k in range(nchunk):
            buf = rows0 if k % 2 == 0 else rows1
            sem = sg0 if k % 2 == 0 else sg1
            r0 = sid * rpt + k * W
            if k >= 2:
                rp = sid * rpt + (k - 2) * W
                pltpu.make_async_copy(buf, out_hbm.at[pl.ds(cid * npad + rp, W)], sem).wait()
            pltpu.sync_copy(acc.at[pl.ds(r0, W)], buf)
            pltpu.make_async_copy(buf, out_hbm.at[pl.ds(cid * npad + r0, W)], sem).start()
        for k in range(max(nchunk - 2, 0), nchunk):
            buf = rows0 if k % 2 == 0 else rows1
            sem = sg0 if k % 2 == 0 else sg1
            r0 = sid * rpt + k * W
            pltpu.make_async_copy(buf, out_hbm.at[pl.ds(cid * npad + r0, W)], sem).wait()

    return prop_kernel


_HI = jax.lax.Precision.HIGHEST


def _tc_proj(n, d, rb):
    def body(x_ref, wp_ref, b_ref, w2e_ref, h0_ref, c_ref):
        h0 = jnp.dot(x_ref[...], wp_ref[...],
                     preferred_element_type=jnp.float32, precision=_HI)
        h0 = h0 + b_ref[...]
        h0_ref[...] = h0
        for i in range(N_LAYERS):
            c_ref[i] = jnp.dot(h0, w2e_ref[i],
                               preferred_element_type=jnp.float32, precision=_HI)

    grid = n // rb
    return pl.pallas_call(
        body,
        grid=(grid,),
        in_specs=[
            pl.BlockSpec((rb, d), lambda i: (i, 0)),
            pl.BlockSpec((d, d), lambda i: (0, 0)),
            pl.BlockSpec((1, d), lambda i: (0, 0)),
            pl.BlockSpec((N_LAYERS, d, d), lambda i: (0, 0, 0)),
        ],
        out_specs=[
            pl.BlockSpec((rb, d), lambda i: (i, 0)),
            pl.BlockSpec((N_LAYERS, rb, d), lambda i: (0, i, 0)),
        ],
        out_shape=[
            jax.ShapeDtypeStruct((n, d), jnp.float32),
            jax.ShapeDtypeStruct((N_LAYERS, n, d), jnp.float32),
        ],
    )


def _tc_scale(n, d, npad, rb):
    def body(deg_ref, h0_ref, g_ref, dis_ref):
        dg = 1.0 + deg_ref[0, :, :1] + deg_ref[1, :, :1]
        dis = jax.lax.rsqrt(dg)
        g_ref[...] = h0_ref[...] * dis
        dis_ref[...] = jnp.broadcast_to(dis, (rb, d))

    return pl.pallas_call(
        body,
        grid=(n // rb,),
        in_specs=[
            pl.BlockSpec((NC, rb, d), lambda i: (0, i, 0)),
            pl.BlockSpec((rb, d), lambda i: (i, 0)),
        ],
        out_specs=[
            pl.BlockSpec((rb, d), lambda i: (i, 0)),
            pl.BlockSpec((rb, d), lambda i: (i, 0)),
        ],
        out_shape=[
            jax.ShapeDtypeStruct((n, d), jnp.float32),
            jax.ShapeDtypeStruct((n, d), jnp.float32),
        ],
    )


def _tc_layer(n, d, npad, rb, emit_g):
    def body(s_ref, g_ref, dis_ref, w1e_ref, c_ref, *out_refs):
        dis = dis_ref[...]
        u = (s_ref[0] + s_ref[1] + g_ref[...]) * dis
        h = jnp.dot(u, w1e_ref[...],
                    preferred_element_type=jnp.float32, precision=_HI)
        h = jnp.maximum(h + c_ref[...], 0.0)
        if emit_g:
            out_refs[0][...] = h * dis
        else:
            out_refs[0][...] = h

    return pl.pallas_call(
        body,
        grid=(n // rb,),
        in_specs=[
            pl.BlockSpec((NC, rb, d), lambda i: (0, i, 0)),
            pl.BlockSpec((rb, d), lambda i: (i, 0)),
            pl.BlockSpec((rb, d), lambda i: (i, 0)),
            pl.BlockSpec((d, d), lambda i: (0, 0)),
            pl.BlockSpec((rb, d), lambda i: (i, 0)),
        ],
        out_specs=pl.BlockSpec((rb, d), lambda i: (i, 0)),
        out_shape=jax.ShapeDtypeStruct((n, d), jnp.float32),
    )


def kernel(x, edge_index, Wproj, bproj, W1, W2):
    n, d = x.shape
    e = edge_index.shape[1]

    ept = -(-e // NW)
    nwin = -(-ept // W)
    nwin = -(-nwin // 8) * 8
    epad = NW * nwin * W
    rpt = -(-(n + 1) // (NS * W)) * W
    npad = NS * rpt
    rb = 1000
    assert n % rb == 0

    src = edge_index[0].astype(jnp.int32)
    dst = edge_index[1].astype(jnp.int32)
    pad = epad - e
    srcp = jnp.concatenate([src, jnp.zeros((pad,), jnp.int32)]).reshape(NW * nwin, W)
    dstp = jnp.concatenate([dst, jnp.full((pad,), n, jnp.int32)]).reshape(NW * nwin, W)
    sd = jnp.stack([srcp, dstp], axis=1)

    eye = jnp.eye(d, dtype=jnp.float32)
    w1e, w2e = [], []
    for i in range(N_LAYERS):
        beta = math.log(THETA / (i + 1) + 1.0)
        w1e.append((1.0 - ALPHA) * ((1.0 - beta) * eye + beta * W1[i]))
        w2e.append(ALPHA * ((1.0 - beta) * eye + beta * W2[i]))
    w1e = jnp.stack(w1e)
    w2e = jnp.stack(w2e)

    zd = jnp.zeros((W, d), jnp.float32)
    onesnd = jnp.ones((n, d), jnp.float32)

    prop = _sc_propagate(n, d, npad, nwin, rpt)

    degd = prop(onesnd, sd, zd).reshape(NC, npad, d)
    h0, c = _tc_proj(n, d, rb)(x, Wproj, bproj.reshape(1, d), w2e)
    g, disrep = _tc_scale(n, d, npad, rb)(degd, h0)

    layer_mid = _tc_layer(n, d, npad, rb, emit_g=True)
    layer_last = _tc_layer(n, d, npad, rb, emit_g=False)
    for i in range(N_LAYERS):
        s = prop(g, sd, zd).reshape(NC, npad, d)
        if i + 1 < N_LAYERS:
            g = layer_mid(s, g, disrep, w1e[i], c[i])
        else:
            h = layer_last(s, g, disrep, w1e[i], c[i])
    return h


# speedup vs baseline: 17.2783x; 2.9716x over previous
import functools
import math

import jax
import jax.numpy as jnp
from jax import lax
from jax.experimental import pallas as pl
from jax.experimental.pallas import tpu as pltpu
from jax.experimental.pallas import tpu_sc as plsc

ALPHA = 0.5
THETA = 1.0
N_LAYERS = 4

NC = 2
NS = 16
NW = NC * NS
W = 128

_MESH = plsc.VectorSubcoreMesh(core_axis_name="c", subcore_axis_name="s")


def _sc_propagate(n, d, npad, nwin, rpt):
    @functools.partial(
        pl.kernel,
        mesh=_MESH,
        out_type=jax.ShapeDtypeStruct((NC * npad, d), jnp.float32),
        scratch_types=[
            pltpu.VMEM((2, W), jnp.int32),
            pltpu.VMEM((2, W), jnp.int32),
            pltpu.VMEM((2, W), jnp.int32),
            pltpu.VMEM((2, W), jnp.int32),
            pltpu.VMEM((2, W), jnp.int32),
            pltpu.VMEM((2, W), jnp.int32),
            pltpu.VMEM((2, W), jnp.int32),
            pltpu.VMEM((2, W), jnp.int32),
            pltpu.VMEM((W, d), jnp.float32),
            pltpu.VMEM((W, d), jnp.float32),
            pltpu.VMEM_SHARED((npad, d), jnp.float32),
            pltpu.SemaphoreType.DMA,
            pltpu.SemaphoreType.DMA,
            pltpu.SemaphoreType.DMA,
            pltpu.SemaphoreType.DMA,
            pltpu.SemaphoreType.DMA,
            pltpu.SemaphoreType.DMA,
            pltpu.SemaphoreType.DMA,
            pltpu.SemaphoreType.DMA,
            pltpu.SemaphoreType.DMA,
            pltpu.SemaphoreType.DMA,
            pltpu.SemaphoreType.DMA,
            pltpu.SemaphoreType.DMA,
        ],
    )
    def prop_kernel(g_hbm, sd_hbm, zeros_hbm, out_hbm,
                    x0, x1, x2, x3, x4, x5, x6, x7, rows0, rows1, acc,
                    i0, i1, i2, i3, i4, i5, i6, i7,
                    sg0, sg1, ss0, ss1):
        slots = (x0, x1, x2, x3, x4, x5, x6, x7)
        isems = (i0, i1, i2, i3, i4, i5, i6, i7)
        rows = (rows0, rows1)
        gsems = (sg0, sg1)
        ssems = (ss0, ss1)
        cid = lax.axis_index("c")
        sid = lax.axis_index("s")
        wid = cid * NS + sid
        base = wid * nwin

        pltpu.sync_copy(zeros_hbm, rows0)

        @pl.loop(0, rpt // W)
        def _(k):
            pltpu.sync_copy(rows0, acc.at[pl.ds(sid * rpt + k * W, W)])

        for k in range(6):
            pltpu.make_async_copy(sd_hbm.at[base + k], slots[k], isems[k]).start()
        pltpu.make_async_copy(sd_hbm.at[base + 0], slots[0], isems[0]).wait()
        pltpu.make_async_copy(g_hbm.at[slots[0].at[0]], rows0, sg0).start()
        pltpu.make_async_copy(sd_hbm.at[base + 1], slots[1], isems[1]).wait()
        pltpu.make_async_copy(g_hbm.at[slots[1].at[0]], rows1, sg1).start()
        plsc.subcore_barrier()

        @pl.loop(0, nwin, step=8)
        def _(j):
            for k in range(0, 8, 2):
                sa, sb = slots[k], slots[k + 1]
                pltpu.make_async_copy(g_hbm.at[sa.at[0]], rows0, sg0).wait()
                pltpu.make_async_copy(
                    rows0, acc.at[sa.at[1]], ss0).start(add=True)
                pltpu.make_async_copy(g_hbm.at[sb.at[0]], rows1, sg1).wait()
                pltpu.make_async_copy(
                    rows1, acc.at[sb.at[1]], ss1).start(add=True)

                @pl.when(j + k + 6 < nwin)
                def _():
                    pltpu.make_async_copy(
                        sd_hbm.at[base + j + k + 6],
                        slots[(k + 6) % 8], isems[(k + 6) % 8]).start()

                @pl.when(j + k + 7 < nwin)
                def _():
                    pltpu.make_async_copy(
                        sd_hbm.at[base + j + k + 7],
                        slots[(k + 7) % 8], isems[(k + 7) % 8]).start()

                pltpu.make_async_copy(rows0, acc.at[sa.at[1]], ss0).wait()

                @pl.when(j + k + 2 < nwin)
                def _():
                    pltpu.make_async_copy(
                        sd_hbm.at[base + j + k + 2],
                        slots[(k + 2) % 8], isems[(k + 2) % 8]).wait()
                    pltpu.make_async_copy(
                        g_hbm.at[slots[(k + 2) % 8].at[0]], rows0, sg0).start()

                pltpu.make_async_copy(rows1, acc.at[sb.at[1]], ss1).wait()

                @pl.when(j + k + 3 < nwin)
                def _():
                    pltpu.make_async_copy(
                        sd_hbm.at[base + j + k + 3],
                        slots[(k + 3) % 8], isems[(k + 3) % 8]).wait()
                    pltpu.make_async_copy(
                        g_hbm.at[slots[(k + 3) % 8].at[0]], rows1, sg1).start()

        plsc.subcore_barrier()

        nchunk = rpt // W
        for k in range(nchunk):
            buf = rows0 if k % 2 == 0 else rows1
            sem = sg0 if k % 2 == 0 else sg1
            r0 = sid * rpt + k * W
            if k >= 2:
                rp = sid * rpt + (k - 2) * W
                pltpu.make_async_copy(buf, out_hbm.at[pl.ds(cid * npad + rp, W)], sem).wait()
            pltpu.sync_copy(acc.at[pl.ds(r0, W)], buf)
            pltpu.make_async_copy(buf, out_hbm.at[pl.ds(cid * npad + r0, W)], sem).start()
        for k in range(max(nchunk - 2, 0), nchunk):
            buf = rows0 if k % 2 == 0 else rows1
            sem = sg0 if k % 2 == 0 else sg1
            r0 = sid * rpt + k * W
            pltpu.make_async_copy(buf, out_hbm.at[pl.ds(cid * npad + r0, W)], sem).wait()

    return prop_kernel


_HI = jax.lax.Precision.HIGHEST


def _tc_proj(n, d, rb):
    def body(x_ref, wp_ref, b_ref, w2e_ref, h0_ref, c_ref):
        h0 = jnp.dot(x_ref[...], wp_ref[...],
                     preferred_element_type=jnp.float32, precision=_HI)
        h0 = h0 + b_ref[...]
        h0_ref[...] = h0
        for i in range(N_LAYERS):
            c_ref[i] = jnp.dot(h0, w2e_ref[i],
                               preferred_element_type=jnp.float32, precision=_HI)

    grid = n // rb
    return pl.pallas_call(
        body,
        grid=(grid,),
        in_specs=[
            pl.BlockSpec((rb, d), lambda i: (i, 0)),
            pl.BlockSpec((d, d), lambda i: (0, 0)),
            pl.BlockSpec((1, d), lambda i: (0, 0)),
            pl.BlockSpec((N_LAYERS, d, d), lambda i: (0, 0, 0)),
        ],
        out_specs=[
            pl.BlockSpec((rb, d), lambda i: (i, 0)),
            pl.BlockSpec((N_LAYERS, rb, d), lambda i: (0, i, 0)),
        ],
        out_shape=[
            jax.ShapeDtypeStruct((n, d), jnp.float32),
            jax.ShapeDtypeStruct((N_LAYERS, n, d), jnp.float32),
        ],
    )


def _tc_scale(n, d, npad, rb):
    def body(deg_ref, h0_ref, g_ref, dis_ref):
        dg = 1.0 + deg_ref[0, :, :1] + deg_ref[1, :, :1]
        dis = jax.lax.rsqrt(dg)
        g_ref[...] = h0_ref[...] * dis
        dis_ref[...] = jnp.broadcast_to(dis, (rb, d))

    return pl.pallas_call(
        body,
        grid=(n // rb,),
        in_specs=[
            pl.BlockSpec((NC, rb, d), lambda i: (0, i, 0)),
            pl.BlockSpec((rb, d), lambda i: (i, 0)),
        ],
        out_specs=[
            pl.BlockSpec((rb, d), lambda i: (i, 0)),
            pl.BlockSpec((rb, d), lambda i: (i, 0)),
        ],
        out_shape=[
            jax.ShapeDtypeStruct((n, d), jnp.float32),
            jax.ShapeDtypeStruct((n, d), jnp.float32),
        ],
    )


def _tc_layer(n, d, npad, rb, emit_g):
    def body(s_ref, g_ref, dis_ref, w1e_ref, c_ref, *out_refs):
        dis = dis_ref[...]
        u = (s_ref[0] + s_ref[1] + g_ref[...]) * dis
        h = jnp.dot(u, w1e_ref[...],
                    preferred_element_type=jnp.float32, precision=_HI)
        h = jnp.maximum(h + c_ref[...], 0.0)
        if emit_g:
            out_refs[0][...] = h * dis
        else:
            out_refs[0][...] = h

    return pl.pallas_call(
        body,
        grid=(n // rb,),
        in_specs=[
            pl.BlockSpec((NC, rb, d), lambda i: (0, i, 0)),
            pl.BlockSpec((rb, d), lambda i: (i, 0)),
            pl.BlockSpec((rb, d), lambda i: (i, 0)),
            pl.BlockSpec((d, d), lambda i: (0, 0)),
            pl.BlockSpec((rb, d), lambda i: (i, 0)),
        ],
        out_specs=pl.BlockSpec((rb, d), lambda i: (i, 0)),
        out_shape=jax.ShapeDtypeStruct((n, d), jnp.float32),
    )


def kernel(x, edge_index, Wproj, bproj, W1, W2):
    n, d = x.shape
    e = edge_index.shape[1]

    ept = -(-e // NW)
    nwin = -(-ept // W)
    nwin = -(-nwin // 8) * 8
    epad = NW * nwin * W
    rpt = -(-(n + 1) // (NS * W)) * W
    npad = NS * rpt
    rb = 1000
    assert n % rb == 0

    src = edge_index[0].astype(jnp.int32)
    dst = edge_index[1].astype(jnp.int32)
    pad = epad - e
    pad_src = jnp.arange(pad, dtype=jnp.int32) % n
    pad_dst = n + (jnp.arange(pad, dtype=jnp.int32) % (npad - n))
    srcp = jnp.concatenate([src, pad_src]).reshape(NW * nwin, W)
    dstp = jnp.concatenate([dst, pad_dst]).reshape(NW * nwin, W)
    sd = jnp.stack([srcp, dstp], axis=1)

    eye = jnp.eye(d, dtype=jnp.float32)
    w1e, w2e = [], []
    for i in range(N_LAYERS):
        beta = math.log(THETA / (i + 1) + 1.0)
        w1e.append((1.0 - ALPHA) * ((1.0 - beta) * eye + beta * W1[i]))
        w2e.append(ALPHA * ((1.0 - beta) * eye + beta * W2[i]))
    w1e = jnp.stack(w1e)
    w2e = jnp.stack(w2e)

    zd = jnp.zeros((W, d), jnp.float32)
    onesnd = jnp.ones((n, d), jnp.float32)

    prop = _sc_propagate(n, d, npad, nwin, rpt)

    degd = prop(onesnd, sd, zd).reshape(NC, npad, d)
    h0, c = _tc_proj(n, d, rb)(x, Wproj, bproj.reshape(1, d), w2e)
    g, disrep = _tc_scale(n, d, npad, rb)(degd, h0)

    layer_mid = _tc_layer(n, d, npad, rb, emit_g=True)
    layer_last = _tc_layer(n, d, npad, rb, emit_g=False)
    for i in range(N_LAYERS):
        s = prop(g, sd, zd).reshape(NC, npad, d)
        if i + 1 < N_LAYERS:
            g = layer_mid(s, g, disrep, w1e[i], c[i])
        else:
            h = layer_last(s, g, disrep, w1e[i], c[i])
    return h


# speedup vs baseline: 18.9461x; 1.0965x over previous
import functools
import math

import jax
import jax.numpy as jnp
from jax import lax
from jax.experimental import pallas as pl
from jax.experimental.pallas import tpu as pltpu
from jax.experimental.pallas import tpu_sc as plsc

ALPHA = 0.5
THETA = 1.0
N_LAYERS = 4

NC = 2
NS = 16
NW = NC * NS
W = 128

_MESH = plsc.VectorSubcoreMesh(core_axis_name="c", subcore_axis_name="s")


def _sc_propagate(n, d, npad, nwin, rpt):
    @functools.partial(
        pl.kernel,
        mesh=_MESH,
        out_type=jax.ShapeDtypeStruct((NC * npad, d), jnp.float32),
        scratch_types=[
            pltpu.VMEM((2, W), jnp.int32),
            pltpu.VMEM((2, W), jnp.int32),
            pltpu.VMEM((2, W), jnp.int32),
            pltpu.VMEM((2, W), jnp.int32),
            pltpu.VMEM((2, W), jnp.int32),
            pltpu.VMEM((2, W), jnp.int32),
            pltpu.VMEM((2, W), jnp.int32),
            pltpu.VMEM((2, W), jnp.int32),
            pltpu.VMEM((W, d), jnp.float32),
            pltpu.VMEM((W, d), jnp.float32),
            pltpu.VMEM_SHARED((npad, d), jnp.float32),
            pltpu.SemaphoreType.DMA,
            pltpu.SemaphoreType.DMA,
            pltpu.SemaphoreType.DMA,
            pltpu.SemaphoreType.DMA,
            pltpu.SemaphoreType.DMA,
            pltpu.SemaphoreType.DMA,
            pltpu.SemaphoreType.DMA,
            pltpu.SemaphoreType.DMA,
            pltpu.SemaphoreType.DMA,
            pltpu.SemaphoreType.DMA,
            pltpu.SemaphoreType.DMA,
            pltpu.SemaphoreType.DMA,
        ],
    )
    def prop_kernel(g_hbm, sd_hbm, zeros_hbm, out_hbm,
                    x0, x1, x2, x3, x4, x5, x6, x7, rows0, rows1, acc,
                    i0, i1, i2, i3, i4, i5, i6, i7,
                    sg0, sg1, ss0, ss1):
        slots = (x0, x1, x2, x3, x4, x5, x6, x7)
        isems = (i0, i1, i2, i3, i4, i5, i6, i7)
        rows = (rows0, rows1)
        gsems = (sg0, sg1)
        ssems = (ss0, ss1)
        cid = lax.axis_index("c")
        sid = lax.axis_index("s")
        wid = cid * NS + sid
        base = wid * nwin

        pltpu.sync_copy(zeros_hbm, rows0)

        @pl.loop(0, rpt // W)
        def _(k):
            pltpu.sync_copy(rows0, acc.at[pl.ds(sid * rpt + k * W, W)])

        for k in range(6):
            pltpu.make_async_copy(sd_hbm.at[base + k], slots[k], isems[k]).start()
        pltpu.make_async_copy(sd_hbm.at[base + 0], slots[0], isems[0]).wait()
        pltpu.make_async_copy(g_hbm.at[slots[0].at[0]], rows0, sg0).start()
        pltpu.make_async_copy(sd_hbm.at[base + 1], slots[1], isems[1]).wait()
        pltpu.make_async_copy(g_hbm.at[slots[1].at[0]], rows1, sg1).start()
        plsc.subcore_barrier()

        @pl.loop(0, nwin, step=8)
        def _(j):
            for k in range(0, 8, 2):
                sa, sb = slots[k], slots[k + 1]
                pltpu.make_async_copy(g_hbm.at[sa.at[0]], rows0, sg0).wait()
                pltpu.make_async_copy(
                    rows0, acc.at[sa.at[1]], ss0).start(add=True)
                pltpu.make_async_copy(g_hbm.at[sb.at[0]], rows1, sg1).wait()
                pltpu.make_async_copy(
                    rows1, acc.at[sb.at[1]], ss1).start(add=True)

                @pl.when(j + k + 6 < nwin)
                def _():
                    pltpu.make_async_copy(
                        sd_hbm.at[base + j + k + 6],
                        slots[(k + 6) % 8], isems[(k + 6) % 8]).start()

                @pl.when(j + k + 7 < nwin)
                def _():
                    pltpu.make_async_copy(
                        sd_hbm.at[base + j + k + 7],
                        slots[(k + 7) % 8], isems[(k + 7) % 8]).start()

                pltpu.make_async_copy(rows0, acc.at[sa.at[1]], ss0).wait()

                @pl.when(j + k + 2 < nwin)
                def _():
                    pltpu.make_async_copy(
                        sd_hbm.at[base + j + k + 2],
                        slots[(k + 2) % 8], isems[(k + 2) % 8]).wait()
                    pltpu.make_async_copy(
                        g_hbm.at[slots[(k + 2) % 8].at[0]], rows0, sg0).start()

                pltpu.make_async_copy(rows1, acc.at[sb.at[1]], ss1).wait()

                @pl.when(j + k + 3 < nwin)
                def _():
                    pltpu.make_async_copy(
                        sd_hbm.at[base + j + k + 3],
                        slots[(k + 3) % 8], isems[(k + 3) % 8]).wait()
                    pltpu.make_async_copy(
                        g_hbm.at[slots[(k + 3) % 8].at[0]], rows1, sg1).start()

        plsc.subcore_barrier()

        nchunk = rpt // W
        for k in range(nchunk):
            buf = rows0 if k % 2 == 0 else rows1
            sem = sg0 if k % 2 == 0 else sg1
            r0 = sid * rpt + k * W
            if k >= 2:
                rp = sid * rpt + (k - 2) * W
                pltpu.make_async_copy(buf, out_hbm.at[pl.ds(cid * npad + rp, W)], sem).wait()
            pltpu.sync_copy(acc.at[pl.ds(r0, W)], buf)
            pltpu.make_async_copy(buf, out_hbm.at[pl.ds(cid * npad + r0, W)], sem).start()
        for k in range(max(nchunk - 2, 0), nchunk):
            buf = rows0 if k % 2 == 0 else rows1
            sem = sg0 if k % 2 == 0 else sg1
            r0 = sid * rpt + k * W
            pltpu.make_async_copy(buf, out_hbm.at[pl.ds(cid * npad + r0, W)], sem).wait()

    return prop_kernel


def _sc_degree(d, npad, nwin, rpt):
    @functools.partial(
        pl.kernel,
        mesh=_MESH,
        out_type=jax.ShapeDtypeStruct((NC * npad, d), jnp.float32),
        scratch_types=[
            pltpu.VMEM((2, W), jnp.int32),
            pltpu.VMEM((2, W), jnp.int32),
            pltpu.VMEM((2, W), jnp.int32),
            pltpu.VMEM((2, W), jnp.int32),
            pltpu.VMEM((2, W), jnp.int32),
            pltpu.VMEM((2, W), jnp.int32),
            pltpu.VMEM((2, W), jnp.int32),
            pltpu.VMEM((2, W), jnp.int32),
            pltpu.VMEM((W, d), jnp.float32),
            pltpu.VMEM((W, d), jnp.float32),
            pltpu.VMEM_SHARED((npad, d), jnp.float32),
            pltpu.SemaphoreType.DMA,
            pltpu.SemaphoreType.DMA,
            pltpu.SemaphoreType.DMA,
            pltpu.SemaphoreType.DMA,
            pltpu.SemaphoreType.DMA,
            pltpu.SemaphoreType.DMA,
            pltpu.SemaphoreType.DMA,
            pltpu.SemaphoreType.DMA,
            pltpu.SemaphoreType.DMA,
            pltpu.SemaphoreType.DMA,
            pltpu.SemaphoreType.DMA,
            pltpu.SemaphoreType.DMA,
            pltpu.SemaphoreType.DMA,
            pltpu.SemaphoreType.DMA,
            pltpu.SemaphoreType.DMA,
            pltpu.SemaphoreType.DMA,
        ],
    )
    def deg_kernel(sd_hbm, ones_hbm, zeros_hbm, out_hbm,
                   x0, x1, x2, x3, x4, x5, x6, x7, rows0, rows1, acc,
                   i0, i1, i2, i3, i4, i5, i6, i7,
                   s0, s1, s2, s3, s4, s5, s6, s7):
        slots = (x0, x1, x2, x3, x4, x5, x6, x7)
        isems = (i0, i1, i2, i3, i4, i5, i6, i7)
        ssems = (s0, s1, s2, s3, s4, s5, s6, s7)
        cid = lax.axis_index("c")
        sid = lax.axis_index("s")
        wid = cid * NS + sid
        base = wid * nwin

        pltpu.sync_copy(zeros_hbm, rows0)

        @pl.loop(0, rpt // W)
        def _(k):
            pltpu.sync_copy(rows0, acc.at[pl.ds(sid * rpt + k * W, W)])

        pltpu.sync_copy(ones_hbm, rows1)
        for k in range(2):
            pltpu.make_async_copy(sd_hbm.at[base + k], slots[k], isems[k]).start()
        plsc.subcore_barrier()

        @pl.loop(0, nwin, step=8)
        def _(j):
            for k in range(8):
                w_off = j + k
                if k >= 6:
                    pltpu.make_async_copy(
                        rows1, acc.at[slots[(k - 6) % 8].at[1]],
                        ssems[(k - 6) % 8]).wait()
                else:
                    @pl.when(j > 0)
                    def _():
                        pltpu.make_async_copy(
                            rows1, acc.at[slots[(k - 6) % 8].at[1]],
                            ssems[(k - 6) % 8]).wait()

                @pl.when(w_off + 2 < nwin)
                def _():
                    pltpu.make_async_copy(
                        sd_hbm.at[base + w_off + 2],
                        slots[(k + 2) % 8], isems[(k + 2) % 8]).start()

                pltpu.make_async_copy(
                    sd_hbm.at[base + w_off], slots[k], isems[k]).wait()
                pltpu.make_async_copy(
                    rows1, acc.at[slots[k].at[1]], ssems[k]).start(add=True)

        for k in range(2, 8):
            pltpu.make_async_copy(
                rows1, acc.at[slots[k].at[1]], ssems[k]).wait()
        plsc.subcore_barrier()

        nchunk = rpt // W
        for k in range(nchunk):
            buf = rows0 if k % 2 == 0 else rows1
            sem = isems[0] if k % 2 == 0 else isems[1]
            r0 = sid * rpt + k * W
            if k >= 2:
                rp = sid * rpt + (k - 2) * W
                pltpu.make_async_copy(buf, out_hbm.at[pl.ds(cid * npad + rp, W)], sem).wait()
            pltpu.sync_copy(acc.at[pl.ds(r0, W)], buf)
            pltpu.make_async_copy(buf, out_hbm.at[pl.ds(cid * npad + r0, W)], sem).start()
        for k in range(max(nchunk - 2, 0), nchunk):
            buf = rows0 if k % 2 == 0 else rows1
            sem = isems[0] if k % 2 == 0 else isems[1]
            r0 = sid * rpt + k * W
            pltpu.make_async_copy(buf, out_hbm.at[pl.ds(cid * npad + r0, W)], sem).wait()

    return deg_kernel


_HI = jax.lax.Precision.HIGHEST


def _tc_proj(n, d, rb):
    def body(x_ref, wp_ref, b_ref, w2e_ref, h0_ref, c_ref):
        h0 = jnp.dot(x_ref[...], wp_ref[...],
                     preferred_element_type=jnp.float32, precision=_HI)
        h0 = h0 + b_ref[...]
        h0_ref[...] = h0
        for i in range(N_LAYERS):
            c_ref[i] = jnp.dot(h0, w2e_ref[i],
                               preferred_element_type=jnp.float32, precision=_HI)

    grid = n // rb
    return pl.pallas_call(
        body,
        grid=(grid,),
        in_specs=[
            pl.BlockSpec((rb, d), lambda i: (i, 0)),
            pl.BlockSpec((d, d), lambda i: (0, 0)),
            pl.BlockSpec((1, d), lambda i: (0, 0)),
            pl.BlockSpec((N_LAYERS, d, d), lambda i: (0, 0, 0)),
        ],
        out_specs=[
            pl.BlockSpec((rb, d), lambda i: (i, 0)),
            pl.BlockSpec((N_LAYERS, rb, d), lambda i: (0, i, 0)),
        ],
        out_shape=[
            jax.ShapeDtypeStruct((n, d), jnp.float32),
            jax.ShapeDtypeStruct((N_LAYERS, n, d), jnp.float32),
        ],
    )


def _tc_scale(n, d, npad, rb):
    def body(deg_ref, h0_ref, g_ref, dis_ref):
        dg = 1.0 + deg_ref[0, :, :1] + deg_ref[1, :, :1]
        dis = jax.lax.rsqrt(dg)
        g_ref[...] = h0_ref[...] * dis
        dis_ref[...] = jnp.broadcast_to(dis, (rb, d))

    return pl.pallas_call(
        body,
        grid=(n // rb,),
        in_specs=[
            pl.BlockSpec((NC, rb, d), lambda i: (0, i, 0)),
            pl.BlockSpec((rb, d), lambda i: (i, 0)),
        ],
        out_specs=[
            pl.BlockSpec((rb, d), lambda i: (i, 0)),
            pl.BlockSpec((rb, d), lambda i: (i, 0)),
        ],
        out_shape=[
            jax.ShapeDtypeStruct((n, d), jnp.float32),
            jax.ShapeDtypeStruct((n, d), jnp.float32),
        ],
    )


def _tc_layer(n, d, npad, rb, emit_g):
    def body(s_ref, g_ref, dis_ref, w1e_ref, c_ref, *out_refs):
        dis = dis_ref[...]
        u = (s_ref[0] + s_ref[1] + g_ref[...]) * dis
        h = jnp.dot(u, w1e_ref[...],
                    preferred_element_type=jnp.float32, precision=_HI)
        h = jnp.maximum(h + c_ref[...], 0.0)
        if emit_g:
            out_refs[0][...] = h * dis
        else:
            out_refs[0][...] = h

    return pl.pallas_call(
        body,
        grid=(n // rb,),
        in_specs=[
            pl.BlockSpec((NC, rb, d), lambda i: (0, i, 0)),
            pl.BlockSpec((rb, d), lambda i: (i, 0)),
            pl.BlockSpec((rb, d), lambda i: (i, 0)),
            pl.BlockSpec((d, d), lambda i: (0, 0)),
            pl.BlockSpec((rb, d), lambda i: (i, 0)),
        ],
        out_specs=pl.BlockSpec((rb, d), lambda i: (i, 0)),
        out_shape=jax.ShapeDtypeStruct((n, d), jnp.float32),
    )


def kernel(x, edge_index, Wproj, bproj, W1, W2):
    n, d = x.shape
    e = edge_index.shape[1]

    ept = -(-e // NW)
    nwin = -(-ept // W)
    nwin = -(-nwin // 8) * 8
    epad = NW * nwin * W
    rpt = -(-(n + 1) // (NS * W)) * W
    npad = NS * rpt
    rb = 1000
    assert n % rb == 0

    src = edge_index[0].astype(jnp.int32)
    dst = edge_index[1].astype(jnp.int32)
    pad = epad - e
    pad_src = jnp.arange(pad, dtype=jnp.int32) % n
    pad_dst = n + (jnp.arange(pad, dtype=jnp.int32) % (npad - n))
    srcp = jnp.concatenate([src, pad_src]).reshape(NW * nwin, W)
    dstp = jnp.concatenate([dst, pad_dst]).reshape(NW * nwin, W)
    sd = jnp.stack([srcp, dstp], axis=1)

    eye = jnp.eye(d, dtype=jnp.float32)
    w1e, w2e = [], []
    for i in range(N_LAYERS):
        beta = math.log(THETA / (i + 1) + 1.0)
        w1e.append((1.0 - ALPHA) * ((1.0 - beta) * eye + beta * W1[i]))
        w2e.append(ALPHA * ((1.0 - beta) * eye + beta * W2[i]))
    w1e = jnp.stack(w1e)
    w2e = jnp.stack(w2e)

    zd = jnp.zeros((W, d), jnp.float32)
    onesd = jnp.ones((W, d), jnp.float32)

    prop = _sc_propagate(n, d, npad, nwin, rpt)

    degd = _sc_degree(d, npad, nwin, rpt)(sd, onesd, zd).reshape(NC, npad, d)
    h0, c = _tc_proj(n, d, rb)(x, Wproj, bproj.reshape(1, d), w2e)
    g, disrep = _tc_scale(n, d, npad, rb)(degd, h0)

    layer_mid = _tc_layer(n, d, npad, rb, emit_g=True)
    layer_last = _tc_layer(n, d, npad, rb, emit_g=False)
    for i in range(N_LAYERS):
        s = prop(g, sd, zd).reshape(NC, npad, d)
        if i + 1 < N_LAYERS:
            g = layer_mid(s, g, disrep, w1e[i], c[i])
        else:
            h = layer_last(s, g, disrep, w1e[i], c[i])
    return h


# speedup vs baseline: 18.9655x; 1.0010x over previous
import functools
import math

import jax
import jax.numpy as jnp
from jax import lax
from jax.experimental import pallas as pl
from jax.experimental.pallas import tpu as pltpu
from jax.experimental.pallas import tpu_sc as plsc

ALPHA = 0.5
THETA = 1.0
N_LAYERS = 4

NC = 2
NS = 16
NW = NC * NS
W = 128

_MESH = plsc.VectorSubcoreMesh(core_axis_name="c", subcore_axis_name="s")


def _sc_propagate(n, d, npad, nwin, rpt):
    @functools.partial(
        pl.kernel,
        mesh=_MESH,
        out_type=jax.ShapeDtypeStruct((NC * npad, d), jnp.float32),
        scratch_types=[
            pltpu.VMEM((2, W), jnp.int32),
            pltpu.VMEM((2, W), jnp.int32),
            pltpu.VMEM((2, W), jnp.int32),
            pltpu.VMEM((2, W), jnp.int32),
            pltpu.VMEM((2, W), jnp.int32),
            pltpu.VMEM((2, W), jnp.int32),
            pltpu.VMEM((2, W), jnp.int32),
            pltpu.VMEM((2, W), jnp.int32),
            pltpu.VMEM((W, d), jnp.float32),
            pltpu.VMEM((W, d), jnp.float32),
            pltpu.VMEM_SHARED((npad, d), jnp.float32),
            pltpu.SemaphoreType.DMA,
            pltpu.SemaphoreType.DMA,
            pltpu.SemaphoreType.DMA,
            pltpu.SemaphoreType.DMA,
            pltpu.SemaphoreType.DMA,
            pltpu.SemaphoreType.DMA,
            pltpu.SemaphoreType.DMA,
            pltpu.SemaphoreType.DMA,
            pltpu.SemaphoreType.DMA,
            pltpu.SemaphoreType.DMA,
            pltpu.SemaphoreType.DMA,
            pltpu.SemaphoreType.DMA,
        ],
    )
    def prop_kernel(g_hbm, sd_hbm, zeros_hbm, out_hbm,
                    x0, x1, x2, x3, x4, x5, x6, x7, rows0, rows1, acc,
                    i0, i1, i2, i3, i4, i5, i6, i7,
                    sg0, sg1, ss0, ss1):
        slots = (x0, x1, x2, x3, x4, x5, x6, x7)
        isems = (i0, i1, i2, i3, i4, i5, i6, i7)
        rows = (rows0, rows1)
        gsems = (sg0, sg1)
        ssems = (ss0, ss1)
        cid = lax.axis_index("c")
        sid = lax.axis_index("s")
        wid = cid * NS + sid
        base = wid * nwin
        H = W // 2

        def gather_start(slot, buf, sem):
            pltpu.make_async_copy(
                g_hbm.at[slot.at[0].at[pl.ds(0, H)]], buf.at[pl.ds(0, H)], sem).start()
            pltpu.make_async_copy(
                g_hbm.at[slot.at[0].at[pl.ds(H, H)]], buf.at[pl.ds(H, H)], sem).start()

        def gather_wait(slot, buf, sem):
            pltpu.make_async_copy(
                g_hbm.at[slot.at[0].at[pl.ds(0, H)]], buf.at[pl.ds(0, H)], sem).wait()
            pltpu.make_async_copy(
                g_hbm.at[slot.at[0].at[pl.ds(H, H)]], buf.at[pl.ds(H, H)], sem).wait()

        pltpu.sync_copy(zeros_hbm, rows0)

        @pl.loop(0, rpt // W)
        def _(k):
            pltpu.sync_copy(rows0, acc.at[pl.ds(sid * rpt + k * W, W)])

        for k in range(6):
            pltpu.make_async_copy(sd_hbm.at[base + k], slots[k], isems[k]).start()
        pltpu.make_async_copy(sd_hbm.at[base + 0], slots[0], isems[0]).wait()
        gather_start(slots[0], rows0, sg0)
        pltpu.make_async_copy(sd_hbm.at[base + 1], slots[1], isems[1]).wait()
        gather_start(slots[1], rows1, sg1)
        plsc.subcore_barrier()

        @pl.loop(0, nwin, step=8)
        def _(j):
            for k in range(0, 8, 2):
                sa, sb = slots[k], slots[k + 1]
                gather_wait(sa, rows0, sg0)
                pltpu.make_async_copy(
                    rows0, acc.at[sa.at[1]], ss0).start(add=True)
                gather_wait(sb, rows1, sg1)
                pltpu.make_async_copy(
                    rows1, acc.at[sb.at[1]], ss1).start(add=True)

                @pl.when(j + k + 6 < nwin)
                def _():
                    pltpu.make_async_copy(
                        sd_hbm.at[base + j + k + 6],
                        slots[(k + 6) % 8], isems[(k + 6) % 8]).start()

                @pl.when(j + k + 7 < nwin)
                def _():
                    pltpu.make_async_copy(
                        sd_hbm.at[base + j + k + 7],
                        slots[(k + 7) % 8], isems[(k + 7) % 8]).start()

                pltpu.make_async_copy(rows0, acc.at[sa.at[1]], ss0).wait()

                @pl.when(j + k + 2 < nwin)
                def _():
                    pltpu.make_async_copy(
                        sd_hbm.at[base + j + k + 2],
                        slots[(k + 2) % 8], isems[(k + 2) % 8]).wait()
                    gather_start(slots[(k + 2) % 8], rows0, sg0)

                pltpu.make_async_copy(rows1, acc.at[sb.at[1]], ss1).wait()

                @pl.when(j + k + 3 < nwin)
                def _():
                    pltpu.make_async_copy(
                        sd_hbm.at[base + j + k + 3],
                        slots[(k + 3) % 8], isems[(k + 3) % 8]).wait()
                    gather_start(slots[(k + 3) % 8], rows1, sg1)

        plsc.subcore_barrier()

        nchunk = rpt // W
        for k in range(nchunk):
            buf = rows0 if k % 2 == 0 else rows1
            sem = sg0 if k % 2 == 0 else sg1
            r0 = sid * rpt + k * W
            if k >= 2:
                rp = sid * rpt + (k - 2) * W
                pltpu.make_async_copy(buf, out_hbm.at[pl.ds(cid * npad + rp, W)], sem).wait()
            pltpu.sync_copy(acc.at[pl.ds(r0, W)], buf)
            pltpu.make_async_copy(buf, out_hbm.at[pl.ds(cid * npad + r0, W)], sem).start()
        for k in range(max(nchunk - 2, 0), nchunk):
            buf = rows0 if k % 2 == 0 else rows1
            sem = sg0 if k % 2 == 0 else sg1
            r0 = sid * rpt + k * W
            pltpu.make_async_copy(buf, out_hbm.at[pl.ds(cid * npad + r0, W)], sem).wait()

    return prop_kernel


def _sc_degree(d, npad, nwin, rpt):
    @functools.partial(
        pl.kernel,
        mesh=_MESH,
        out_type=jax.ShapeDtypeStruct((NC * npad, d), jnp.float32),
        scratch_types=[
            pltpu.VMEM((2, W), jnp.int32),
            pltpu.VMEM((2, W), jnp.int32),
            pltpu.VMEM((2, W), jnp.int32),
            pltpu.VMEM((2, W), jnp.int32),
            pltpu.VMEM((2, W), jnp.int32),
            pltpu.VMEM((2, W), jnp.int32),
            pltpu.VMEM((2, W), jnp.int32),
            pltpu.VMEM((2, W), jnp.int32),
            pltpu.VMEM((W, d), jnp.float32),
            pltpu.VMEM((W, d), jnp.float32),
            pltpu.VMEM_SHARED((npad, d), jnp.float32),
            pltpu.SemaphoreType.DMA,
            pltpu.SemaphoreType.DMA,
            pltpu.SemaphoreType.DMA,
            pltpu.SemaphoreType.DMA,
            pltpu.SemaphoreType.DMA,
            pltpu.SemaphoreType.DMA,
            pltpu.SemaphoreType.DMA,
            pltpu.SemaphoreType.DMA,
            pltpu.SemaphoreType.DMA,
            pltpu.SemaphoreType.DMA,
            pltpu.SemaphoreType.DMA,
            pltpu.SemaphoreType.DMA,
            pltpu.SemaphoreType.DMA,
            pltpu.SemaphoreType.DMA,
            pltpu.SemaphoreType.DMA,
            pltpu.SemaphoreType.DMA,
        ],
    )
    def deg_kernel(sd_hbm, ones_hbm, zeros_hbm, out_hbm,
                   x0, x1, x2, x3, x4, x5, x6, x7, rows0, rows1, acc,
                   i0, i1, i2, i3, i4, i5, i6, i7,
                   s0, s1, s2, s3, s4, s5, s6, s7):
        slots = (x0, x1, x2, x3, x4, x5, x6, x7)
        isems = (i0, i1, i2, i3, i4, i5, i6, i7)
        ssems = (s0, s1, s2, s3, s4, s5, s6, s7)
        cid = lax.axis_index("c")
        sid = lax.axis_index("s")
        wid = cid * NS + sid
        base = wid * nwin

        pltpu.sync_copy(zeros_hbm, rows0)

        @pl.loop(0, rpt // W)
        def _(k):
            pltpu.sync_copy(rows0, acc.at[pl.ds(sid * rpt + k * W, W)])

        pltpu.sync_copy(ones_hbm, rows1)
        for k in range(2):
            pltpu.make_async_copy(sd_hbm.at[base + k], slots[k], isems[k]).start()
        plsc.subcore_barrier()

        @pl.loop(0, nwin, step=8)
        def _(j):
            for k in range(8):
                w_off = j + k
                if k >= 6:
                    pltpu.make_async_copy(
                        rows1, acc.at[slots[(k - 6) % 8].at[1]],
                        ssems[(k - 6) % 8]).wait()
                else:
                    @pl.when(j > 0)
                    def _():
                        pltpu.make_async_copy(
                            rows1, acc.at[slots[(k - 6) % 8].at[1]],
                            ssems[(k - 6) % 8]).wait()

                @pl.when(w_off + 2 < nwin)
                def _():
                    pltpu.make_async_copy(
                        sd_hbm.at[base + w_off + 2],
                        slots[(k + 2) % 8], isems[(k + 2) % 8]).start()

                pltpu.make_async_copy(
                    sd_hbm.at[base + w_off], slots[k], isems[k]).wait()
                pltpu.make_async_copy(
                    rows1, acc.at[slots[k].at[1]], ssems[k]).start(add=True)

        for k in range(2, 8):
            pltpu.make_async_copy(
                rows1, acc.at[slots[k].at[1]], ssems[k]).wait()
        plsc.subcore_barrier()

        nchunk = rpt // W
        for k in range(nchunk):
            buf = rows0 if k % 2 == 0 else rows1
            sem = isems[0] if k % 2 == 0 else isems[1]
            r0 = sid * rpt + k * W
            if k >= 2:
                rp = sid * rpt + (k - 2) * W
                pltpu.make_async_copy(buf, out_hbm.at[pl.ds(cid * npad + rp, W)], sem).wait()
            pltpu.sync_copy(acc.at[pl.ds(r0, W)], buf)
            pltpu.make_async_copy(buf, out_hbm.at[pl.ds(cid * npad + r0, W)], sem).start()
        for k in range(max(nchunk - 2, 0), nchunk):
            buf = rows0 if k % 2 == 0 else rows1
            sem = isems[0] if k % 2 == 0 else isems[1]
            r0 = sid * rpt + k * W
            pltpu.make_async_copy(buf, out_hbm.at[pl.ds(cid * npad + r0, W)], sem).wait()

    return deg_kernel


_HI = jax.lax.Precision.HIGHEST


def _tc_proj(n, d, rb):
    def body(x_ref, wp_ref, b_ref, w2e_ref, h0_ref, c_ref):
        h0 = jnp.dot(x_ref[...], wp_ref[...],
                     preferred_element_type=jnp.float32, precision=_HI)
        h0 = h0 + b_ref[...]
        h0_ref[...] = h0
        for i in range(N_LAYERS):
            c_ref[i] = jnp.dot(h0, w2e_ref[i],
                               preferred_element_type=jnp.float32, precision=_HI)

    grid = n // rb
    return pl.pallas_call(
        body,
        grid=(grid,),
        in_specs=[
            pl.BlockSpec((rb, d), lambda i: (i, 0)),
            pl.BlockSpec((d, d), lambda i: (0, 0)),
            pl.BlockSpec((1, d), lambda i: (0, 0)),
            pl.BlockSpec((N_LAYERS, d, d), lambda i: (0, 0, 0)),
        ],
        out_specs=[
            pl.BlockSpec((rb, d), lambda i: (i, 0)),
            pl.BlockSpec((N_LAYERS, rb, d), lambda i: (0, i, 0)),
        ],
        out_shape=[
            jax.ShapeDtypeStruct((n, d), jnp.float32),
            jax.ShapeDtypeStruct((N_LAYERS, n, d), jnp.float32),
        ],
    )


def _tc_scale(n, d, npad, rb):
    def body(deg_ref, h0_ref, g_ref, dis_ref):
        dg = 1.0 + deg_ref[0, :, :1] + deg_ref[1, :, :1]
        dis = jax.lax.rsqrt(dg)
        g_ref[...] = h0_ref[...] * dis
        dis_ref[...] = jnp.broadcast_to(dis, (rb, d))

    return pl.pallas_call(
        body,
        grid=(n // rb,),
        in_specs=[
            pl.BlockSpec((NC, rb, d), lambda i: (0, i, 0)),
            pl.BlockSpec((rb, d), lambda i: (i, 0)),
        ],
        out_specs=[
            pl.BlockSpec((rb, d), lambda i: (i, 0)),
            pl.BlockSpec((rb, d), lambda i: (i, 0)),
        ],
        out_shape=[
            jax.ShapeDtypeStruct((n, d), jnp.float32),
            jax.ShapeDtypeStruct((n, d), jnp.float32),
        ],
    )


def _tc_layer(n, d, npad, rb, emit_g):
    def body(s_ref, g_ref, dis_ref, w1e_ref, c_ref, *out_refs):
        dis = dis_ref[...]
        u = (s_ref[0] + s_ref[1] + g_ref[...]) * dis
        h = jnp.dot(u, w1e_ref[...],
                    preferred_element_type=jnp.float32, precision=_HI)
        h = jnp.maximum(h + c_ref[...], 0.0)
        if emit_g:
            out_refs[0][...] = h * dis
        else:
            out_refs[0][...] = h

    return pl.pallas_call(
        body,
        grid=(n // rb,),
        in_specs=[
            pl.BlockSpec((NC, rb, d), lambda i: (0, i, 0)),
            pl.BlockSpec((rb, d), lambda i: (i, 0)),
            pl.BlockSpec((rb, d), lambda i: (i, 0)),
            pl.BlockSpec((d, d), lambda i: (0, 0)),
            pl.BlockSpec((rb, d), lambda i: (i, 0)),
        ],
        out_specs=pl.BlockSpec((rb, d), lambda i: (i, 0)),
        out_shape=jax.ShapeDtypeStruct((n, d), jnp.float32),
    )


def kernel(x, edge_index, Wproj, bproj, W1, W2):
    n, d = x.shape
    e = edge_index.shape[1]

    ept = -(-e // NW)
    nwin = -(-ept // W)
    nwin = -(-nwin // 8) * 8
    epad = NW * nwin * W
    rpt = -(-(n + 1) // (NS * W)) * W
    npad = NS * rpt
    rb = 1000
    assert n % rb == 0

    src = edge_index[0].astype(jnp.int32)
    dst = edge_index[1].astype(jnp.int32)
    pad = epad - e
    pad_src = jnp.arange(pad, dtype=jnp.int32) % n
    pad_dst = n + (jnp.arange(pad, dtype=jnp.int32) % (npad - n))
    srcp = jnp.concatenate([src, pad_src]).reshape(NW * nwin, W)
    dstp = jnp.concatenate([dst, pad_dst]).reshape(NW * nwin, W)
    sd = jnp.stack([srcp, dstp], axis=1)

    eye = jnp.eye(d, dtype=jnp.float32)
    w1e, w2e = [], []
    for i in range(N_LAYERS):
        beta = math.log(THETA / (i + 1) + 1.0)
        w1e.append((1.0 - ALPHA) * ((1.0 - beta) * eye + beta * W1[i]))
        w2e.append(ALPHA * ((1.0 - beta) * eye + beta * W2[i]))
    w1e = jnp.stack(w1e)
    w2e = jnp.stack(w2e)

    zd = jnp.zeros((W, d), jnp.float32)
    onesd = jnp.ones((W, d), jnp.float32)

    prop = _sc_propagate(n, d, npad, nwin, rpt)

    degd = _sc_degree(d, npad, nwin, rpt)(sd, onesd, zd).reshape(NC, npad, d)
    h0, c = _tc_proj(n, d, rb)(x, Wproj, bproj.reshape(1, d), w2e)
    g, disrep = _tc_scale(n, d, npad, rb)(degd, h0)

    layer_mid = _tc_layer(n, d, npad, rb, emit_g=True)
    layer_last = _tc_layer(n, d, npad, rb, emit_g=False)
    for i in range(N_LAYERS):
        s = prop(g, sd, zd).reshape(NC, npad, d)
        if i + 1 < N_LAYERS:
            g = layer_mid(s, g, disrep, w1e[i], c[i])
        else:
            h = layer_last(s, g, disrep, w1e[i], c[i])
    return h
